# Initial kernel scaffold; baseline (speedup 1.0000x reference)
import jax
import jax.numpy as jnp
from jax.experimental import pallas as pl


def kernel(x, edge_index, W1, b1, W2, b2):
    raise NotImplementedError("write your pallas kernel here")


# speedup vs baseline: 16.2839x; 16.2839x over previous
import functools

import jax
import jax.numpy as jnp
from jax import lax
from jax.experimental import pallas as pl
from jax.experimental.pallas import tpu as pltpu
from jax.experimental.pallas import tpu_sc as plsc

NN = 10000
EE = 320000
DH = 16

NC = 2
NS = 16
CH = 80
EPT = EE // (NC * NS)
NCHUNK = EPT // CH
RPT = NN // NS

_mesh = plsc.VectorSubcoreMesh(core_axis_name="c", subcore_axis_name="s")


def _zero_fill(buf, nrows):
    def body(i, _):
        buf[i, :] = jnp.zeros((DH,), jnp.float32)
        return _
    lax.fori_loop(0, nrows, body, None)


def _deg_body(dst_hbm, out_hbm, idx_d, ones_v, stage, zsh):
    cid = lax.axis_index("c")
    sid = lax.axis_index("s")

    def fill_ones(i, _):
        ones_v[i, :] = jnp.full((DH,), 1.0, jnp.float32)
        return _
    lax.fori_loop(0, CH, fill_ones, None)

    _zero_fill(stage, RPT)
    pltpu.sync_copy(stage, zsh.at[pl.ds(sid * RPT, RPT)])
    plsc.subcore_barrier()

    base = (cid * NS + sid) * EPT

    def chunk(i, _):
        pltpu.sync_copy(dst_hbm.at[pl.ds(base + i * CH, CH)], idx_d)
        pltpu.sync_copy(ones_v, zsh.at[idx_d], add=True)
        return _
    lax.fori_loop(0, NCHUNK, chunk, None)
    plsc.subcore_barrier()

    pltpu.sync_copy(zsh.at[pl.ds(sid * RPT, RPT)], stage)
    pltpu.sync_copy(stage, out_hbm.at[cid, sid])


_deg_call = pl.kernel(
    _deg_body,
    out_type=jax.ShapeDtypeStruct((NC, NS, RPT, DH), jnp.float32),
    mesh=_mesh,
    compiler_params=pltpu.CompilerParams(use_tc_tiling_on_sc=False),
    scratch_types=[
        pltpu.VMEM((CH,), jnp.int32),
        pltpu.VMEM((CH, DH), jnp.float32),
        pltpu.VMEM((RPT, DH), jnp.float32),
        pltpu.VMEM_SHARED((NN, DH), jnp.float32),
    ],
)


def _scat_body(y_hbm, src_hbm, dst_hbm, out_hbm, idx_s, idx_d, rows, stage, zsh,
               sem):
    cid = lax.axis_index("c")
    sid = lax.axis_index("s")

    _zero_fill(stage, RPT)
    pltpu.sync_copy(stage, zsh.at[pl.ds(sid * RPT, RPT)])
    plsc.subcore_barrier()

    base = (cid * NS + sid) * EPT

    def chunk(i, _):
        pltpu.sync_copy(src_hbm.at[pl.ds(base + i * CH, CH)], idx_s)
        pltpu.sync_copy(dst_hbm.at[pl.ds(base + i * CH, CH)], idx_d)
        pltpu.async_copy(y_hbm.at[idx_s], rows, sem).wait()
        pltpu.sync_copy(rows, zsh.at[idx_d], add=True)
        return _
    lax.fori_loop(0, NCHUNK, chunk, None)
    plsc.subcore_barrier()

    pltpu.sync_copy(zsh.at[pl.ds(sid * RPT, RPT)], stage)
    pltpu.sync_copy(stage, out_hbm.at[cid, sid])


_scat_call = pl.kernel(
    _scat_body,
    out_type=jax.ShapeDtypeStruct((NC, NS, RPT, DH), jnp.float32),
    mesh=_mesh,
    compiler_params=pltpu.CompilerParams(use_tc_tiling_on_sc=False),
    scratch_types=[
        pltpu.VMEM((CH,), jnp.int32),
        pltpu.VMEM((CH,), jnp.int32),
        pltpu.VMEM((CH, DH), jnp.float32),
        pltpu.VMEM((RPT, DH), jnp.float32),
        pltpu.VMEM_SHARED((NN, DH), jnp.float32),
        pltpu.SemaphoreType.DMA,
    ],
)


def _prep_body(x_ref, w1_ref, degp_ref, y1_ref, dinv_ref):
    xw = jnp.dot(x_ref[...], w1_ref[...], preferred_element_type=jnp.float32)
    deg = degp_ref[0] + degp_ref[1] + 1.0
    dinv = lax.rsqrt(deg)
    dinv_ref[...] = dinv
    y1_ref[...] = dinv * xw


_prep_call = pl.pallas_call(
    _prep_body,
    out_shape=[
        jax.ShapeDtypeStruct((NN, DH), jnp.float32),
        jax.ShapeDtypeStruct((NN, DH), jnp.float32),
    ],
)


def _mid_body(y1_ref, z_ref, dinv_ref, b1_ref, y2_ref):
    z = y1_ref[...] + z_ref[0] + z_ref[1]
    h = jnp.maximum(dinv_ref[...] * z + b1_ref[...], 0.0)
    y2_ref[...] = dinv_ref[...] * h


_mid_call = pl.pallas_call(
    _mid_body,
    out_shape=jax.ShapeDtypeStruct((NN, DH), jnp.float32),
)


def _fin_body(y2_ref, z_ref, dinv_ref, w2_ref, b2_ref, o_ref):
    t = dinv_ref[...] * (y2_ref[...] + z_ref[0] + z_ref[1])
    h = jnp.dot(t, w2_ref[...], preferred_element_type=jnp.float32) + b2_ref[...]
    m = jnp.max(h, axis=1, keepdims=True)
    s = h - m
    lse = jnp.log(jnp.sum(jnp.exp(s), axis=1, keepdims=True))
    o_ref[...] = s - lse


_fin_call = pl.pallas_call(
    _fin_body,
    out_shape=jax.ShapeDtypeStruct((NN, 2), jnp.float32),
)


def kernel(x, edge_index, W1, b1, W2, b2):
    src = edge_index[0]
    dst = edge_index[1]
    degp = _deg_call(dst).reshape(NC, NN, DH)
    y1, dinv = _prep_call(x, W1, degp)
    z1 = _scat_call(y1, src, dst).reshape(NC, NN, DH)
    y2 = _mid_call(y1, z1, dinv, b1.reshape(1, DH))
    z2 = _scat_call(y2, src, dst).reshape(NC, NN, DH)
    return _fin_call(y2, z2, dinv, W2, b2.reshape(1, 2))


# speedup vs baseline: 31.3637x; 1.9261x over previous
import functools

import jax
import jax.numpy as jnp
from jax import lax
from jax.experimental import pallas as pl
from jax.experimental.pallas import tpu as pltpu
from jax.experimental.pallas import tpu_sc as plsc

NN = 10000
EE = 320000
DH = 16

NC = 2
NS = 16
CH = 80
EPT = EE // (NC * NS)
NCHUNK = EPT // CH
RPT = NN // NS

_mesh = plsc.VectorSubcoreMesh(core_axis_name="c", subcore_axis_name="s")


def _zero_fill(buf, nrows):
    def body(i, _):
        buf[i, :] = jnp.zeros((DH,), jnp.float32)
        return _
    lax.fori_loop(0, nrows, body, None)


def _deg_body(dst_hbm, out_hbm, idx_d, ones_v, stage, zsh):
    cid = lax.axis_index("c")
    sid = lax.axis_index("s")
    wid = cid * NS + sid

    def fill_ones(i, _):
        ones_v[i, :] = jnp.full((DH,), 1.0, jnp.float32)
        return _
    lax.fori_loop(0, CH, fill_ones, None)

    _zero_fill(stage, RPT)
    pltpu.sync_copy(stage, zsh.at[pl.ds(sid * RPT, RPT)])
    pltpu.sync_copy(dst_hbm.at[wid], idx_d)
    plsc.subcore_barrier()

    def chunk(i, _):
        pltpu.sync_copy(ones_v, zsh.at[idx_d.at[i]], add=True)
        return _
    lax.fori_loop(0, NCHUNK, chunk, None)
    plsc.subcore_barrier()

    pltpu.sync_copy(zsh.at[pl.ds(sid * RPT, RPT)], stage)
    pltpu.sync_copy(stage, out_hbm.at[cid, sid])


_deg_call = pl.kernel(
    _deg_body,
    out_type=jax.ShapeDtypeStruct((NC, NS, RPT, DH), jnp.float32),
    mesh=_mesh,
    compiler_params=pltpu.CompilerParams(use_tc_tiling_on_sc=False),
    scratch_types=[
        pltpu.VMEM((NCHUNK, CH), jnp.int32),
        pltpu.VMEM((CH, DH), jnp.float32),
        pltpu.VMEM((RPT, DH), jnp.float32),
        pltpu.VMEM_SHARED((NN, DH), jnp.float32),
    ],
)


def _scat_body(y_hbm, src_hbm, dst_hbm, out_hbm, idx_s, idx_d, rows0, rows1,
               stage, zsh, sem0, sem1):
    cid = lax.axis_index("c")
    sid = lax.axis_index("s")
    wid = cid * NS + sid

    _zero_fill(stage, RPT)
    pltpu.sync_copy(stage, zsh.at[pl.ds(sid * RPT, RPT)])
    pltpu.sync_copy(src_hbm.at[wid], idx_s)
    pltpu.sync_copy(dst_hbm.at[wid], idx_d)
    plsc.subcore_barrier()

    pltpu.async_copy(y_hbm.at[idx_s.at[0]], rows0, sem0)

    def pair(j, _):
        i0 = 2 * j
        pltpu.make_async_copy(y_hbm.at[idx_s.at[i0]], rows0, sem0).wait()
        pltpu.async_copy(y_hbm.at[idx_s.at[i0 + 1]], rows1, sem1)
        pltpu.sync_copy(rows0, zsh.at[idx_d.at[i0]], add=True)
        pltpu.make_async_copy(y_hbm.at[idx_s.at[i0 + 1]], rows1, sem1).wait()
        pltpu.async_copy(y_hbm.at[idx_s.at[i0 + 2]], rows0, sem0)
        pltpu.sync_copy(rows1, zsh.at[idx_d.at[i0 + 1]], add=True)
        return _
    lax.fori_loop(0, (NCHUNK - 1) // 2, pair, None)

    pltpu.make_async_copy(y_hbm.at[idx_s.at[NCHUNK - 1]], rows0, sem0).wait()
    pltpu.sync_copy(rows0, zsh.at[idx_d.at[NCHUNK - 1]], add=True)
    plsc.subcore_barrier()

    pltpu.sync_copy(zsh.at[pl.ds(sid * RPT, RPT)], stage)
    pltpu.sync_copy(stage, out_hbm.at[cid, sid])


_scat_call = pl.kernel(
    _scat_body,
    out_type=jax.ShapeDtypeStruct((NC, NS, RPT, DH), jnp.float32),
    mesh=_mesh,
    compiler_params=pltpu.CompilerParams(use_tc_tiling_on_sc=False),
    scratch_types=[
        pltpu.VMEM((NCHUNK, CH), jnp.int32),
        pltpu.VMEM((NCHUNK, CH), jnp.int32),
        pltpu.VMEM((CH, DH), jnp.float32),
        pltpu.VMEM((CH, DH), jnp.float32),
        pltpu.VMEM((RPT, DH), jnp.float32),
        pltpu.VMEM_SHARED((NN, DH), jnp.float32),
        pltpu.SemaphoreType.DMA,
        pltpu.SemaphoreType.DMA,
    ],
)


def _prep_body(x_ref, w1_ref, degp_ref, y1_ref, dinv_ref):
    xw = jnp.dot(x_ref[...], w1_ref[...], preferred_element_type=jnp.float32)
    deg = degp_ref[0] + degp_ref[1] + 1.0
    dinv = lax.rsqrt(deg)
    dinv_ref[...] = dinv
    y1_ref[...] = dinv * xw


_prep_call = pl.pallas_call(
    _prep_body,
    out_shape=[
        jax.ShapeDtypeStruct((NN, DH), jnp.float32),
        jax.ShapeDtypeStruct((NN, DH), jnp.float32),
    ],
)


def _mid_body(y1_ref, z_ref, dinv_ref, b1_ref, y2_ref):
    z = y1_ref[...] + z_ref[0] + z_ref[1]
    h = jnp.maximum(dinv_ref[...] * z + b1_ref[...], 0.0)
    y2_ref[...] = dinv_ref[...] * h


_mid_call = pl.pallas_call(
    _mid_body,
    out_shape=jax.ShapeDtypeStruct((NN, DH), jnp.float32),
)


def _fin_body(y2_ref, z_ref, dinv_ref, w2_ref, b2_ref, o_ref):
    t = dinv_ref[...] * (y2_ref[...] + z_ref[0] + z_ref[1])
    h = jnp.dot(t, w2_ref[...], preferred_element_type=jnp.float32) + b2_ref[...]
    m = jnp.max(h, axis=1, keepdims=True)
    s = h - m
    lse = jnp.log(jnp.sum(jnp.exp(s), axis=1, keepdims=True))
    o_ref[...] = s - lse


_fin_call = pl.pallas_call(
    _fin_body,
    out_shape=jax.ShapeDtypeStruct((NN, 2), jnp.float32),
)


def kernel(x, edge_index, W1, b1, W2, b2):
    src = edge_index[0].reshape(NC * NS, NCHUNK, CH)
    dst = edge_index[1].reshape(NC * NS, NCHUNK, CH)
    degp = _deg_call(dst).reshape(NC, NN, DH)
    y1, dinv = _prep_call(x, W1, degp)
    z1 = _scat_call(y1, src, dst).reshape(NC, NN, DH)
    y2 = _mid_call(y1, z1, dinv, b1.reshape(1, DH))
    z2 = _scat_call(y2, src, dst).reshape(NC, NN, DH)
    return _fin_call(y2, z2, dinv, W2, b2.reshape(1, 2))


# speedup vs baseline: 53.7012x; 1.7122x over previous
import functools

import jax
import jax.numpy as jnp
from jax import lax
from jax.experimental import pallas as pl
from jax.experimental.pallas import tpu as pltpu
from jax.experimental.pallas import tpu_sc as plsc

NN = 10000
EE = 320000
DH = 16

NC = 2
NS = 16
CH = 80
EPT = EE // (NC * NS)
NCHUNK = EPT // CH
RPT = NN // NS

_mesh = plsc.VectorSubcoreMesh(core_axis_name="c", subcore_axis_name="s")


def _zero_fill(buf, nrows):
    def body(i, _):
        buf[i, :] = jnp.zeros((DH,), jnp.float32)
        return _
    lax.fori_loop(0, nrows, body, None)


def _deg_body(dst_hbm, out_hbm, idx_d, ones_v, stage, zsh):
    cid = lax.axis_index("c")
    sid = lax.axis_index("s")
    wid = cid * NS + sid

    def fill_ones(i, _):
        ones_v[i, :] = jnp.full((DH,), 1.0, jnp.float32)
        return _
    lax.fori_loop(0, CH, fill_ones, None)

    _zero_fill(stage, RPT)
    pltpu.sync_copy(stage, zsh.at[pl.ds(sid * RPT, RPT)])
    pltpu.sync_copy(dst_hbm.at[wid], idx_d)
    plsc.subcore_barrier()

    def chunk(i, _):
        pltpu.sync_copy(ones_v, zsh.at[idx_d.at[i]], add=True)
        return _
    lax.fori_loop(0, NCHUNK, chunk, None)
    plsc.subcore_barrier()

    pltpu.sync_copy(zsh.at[pl.ds(sid * RPT, RPT)], stage)
    pltpu.sync_copy(stage, out_hbm.at[cid, sid])


_deg_call = pl.kernel(
    _deg_body,
    out_type=jax.ShapeDtypeStruct((NC, NS, RPT, DH), jnp.float32),
    mesh=_mesh,
    compiler_params=pltpu.CompilerParams(use_tc_tiling_on_sc=False),
    scratch_types=[
        pltpu.VMEM((NCHUNK, CH), jnp.int32),
        pltpu.VMEM((CH, DH), jnp.float32),
        pltpu.VMEM((RPT, DH), jnp.float32),
        pltpu.VMEM_SHARED((NN, DH), jnp.float32),
    ],
)


def _scat_body(y_hbm, src_hbm, dst_hbm, out_hbm, idx_s, idx_d, rows0, rows1,
               stage, zsh, ybuf, sem0, sem1):
    cid = lax.axis_index("c")
    sid = lax.axis_index("s")
    wid = cid * NS + sid

    _zero_fill(stage, RPT)
    pltpu.sync_copy(stage, zsh.at[pl.ds(sid * RPT, RPT)])
    pltpu.sync_copy(y_hbm.at[pl.ds(sid * RPT, RPT)],
                    ybuf.at[pl.ds(sid * RPT, RPT)])
    pltpu.sync_copy(src_hbm.at[wid], idx_s)
    pltpu.sync_copy(dst_hbm.at[wid], idx_d)
    plsc.subcore_barrier()

    pltpu.async_copy(ybuf.at[idx_s.at[0]], rows0, sem0)

    def pair(j, _):
        i0 = 2 * j
        pltpu.make_async_copy(ybuf.at[idx_s.at[i0]], rows0, sem0).wait()
        pltpu.async_copy(ybuf.at[idx_s.at[i0 + 1]], rows1, sem1)
        pltpu.sync_copy(rows0, zsh.at[idx_d.at[i0]], add=True)
        pltpu.make_async_copy(ybuf.at[idx_s.at[i0 + 1]], rows1, sem1).wait()
        pltpu.async_copy(ybuf.at[idx_s.at[i0 + 2]], rows0, sem0)
        pltpu.sync_copy(rows1, zsh.at[idx_d.at[i0 + 1]], add=True)
        return _
    lax.fori_loop(0, (NCHUNK - 1) // 2, pair, None)

    pltpu.make_async_copy(ybuf.at[idx_s.at[NCHUNK - 1]], rows0, sem0).wait()
    pltpu.sync_copy(rows0, zsh.at[idx_d.at[NCHUNK - 1]], add=True)
    plsc.subcore_barrier()

    pltpu.sync_copy(zsh.at[pl.ds(sid * RPT, RPT)], stage)
    pltpu.sync_copy(stage, out_hbm.at[cid, sid])


_scat_call = pl.kernel(
    _scat_body,
    out_type=jax.ShapeDtypeStruct((NC, NS, RPT, DH), jnp.float32),
    mesh=_mesh,
    compiler_params=pltpu.CompilerParams(use_tc_tiling_on_sc=False),
    scratch_types=[
        pltpu.VMEM((NCHUNK, CH), jnp.int32),
        pltpu.VMEM((NCHUNK, CH), jnp.int32),
        pltpu.VMEM((CH, DH), jnp.float32),
        pltpu.VMEM((CH, DH), jnp.float32),
        pltpu.VMEM((RPT, DH), jnp.float32),
        pltpu.VMEM_SHARED((NN, DH), jnp.float32),
        pltpu.VMEM_SHARED((NN, DH), jnp.float32),
        pltpu.SemaphoreType.DMA,
        pltpu.SemaphoreType.DMA,
    ],
)


def _prep_body(x_ref, w1_ref, degp_ref, y1_ref, dinv_ref):
    xw = jnp.dot(x_ref[...], w1_ref[...], preferred_element_type=jnp.float32)
    deg = degp_ref[0] + degp_ref[1] + 1.0
    dinv = lax.rsqrt(deg)
    dinv_ref[...] = dinv
    y1_ref[...] = dinv * xw


_prep_call = pl.pallas_call(
    _prep_body,
    out_shape=[
        jax.ShapeDtypeStruct((NN, DH), jnp.float32),
        jax.ShapeDtypeStruct((NN, DH), jnp.float32),
    ],
)


def _mid_body(y1_ref, z_ref, dinv_ref, b1_ref, y2_ref):
    z = y1_ref[...] + z_ref[0] + z_ref[1]
    h = jnp.maximum(dinv_ref[...] * z + b1_ref[...], 0.0)
    y2_ref[...] = dinv_ref[...] * h


_mid_call = pl.pallas_call(
    _mid_body,
    out_shape=jax.ShapeDtypeStruct((NN, DH), jnp.float32),
)


def _fin_body(y2_ref, z_ref, dinv_ref, w2_ref, b2_ref, o_ref):
    t = dinv_ref[...] * (y2_ref[...] + z_ref[0] + z_ref[1])
    h = jnp.dot(t, w2_ref[...], preferred_element_type=jnp.float32) + b2_ref[...]
    m = jnp.max(h, axis=1, keepdims=True)
    s = h - m
    lse = jnp.log(jnp.sum(jnp.exp(s), axis=1, keepdims=True))
    o_ref[...] = s - lse


_fin_call = pl.pallas_call(
    _fin_body,
    out_shape=jax.ShapeDtypeStruct((NN, 2), jnp.float32),
)


def kernel(x, edge_index, W1, b1, W2, b2):
    src = edge_index[0].reshape(NC * NS, NCHUNK, CH)
    dst = edge_index[1].reshape(NC * NS, NCHUNK, CH)
    degp = _deg_call(dst).reshape(NC, NN, DH)
    y1, dinv = _prep_call(x, W1, degp)
    z1 = _scat_call(y1, src, dst).reshape(NC, NN, DH)
    y2 = _mid_call(y1, z1, dinv, b1.reshape(1, DH))
    z2 = _scat_call(y2, src, dst).reshape(NC, NN, DH)
    return _fin_call(y2, z2, dinv, W2, b2.reshape(1, 2))
